# baseline (speedup 1.0000x reference)
import functools

import jax
import jax.numpy as jnp
from jax.experimental import pallas as pl
from jax.experimental.pallas import tpu as pltpu
from jax.experimental.pallas import tpu_sc as plsc

_B, _S, _H = 4, 2048, 2048
_E, _K, _R, _RH = 8, 2, 16, 256
_SCALE = 32.0 / 16.0
_BAL_W = 0.1
_NEG = -1e30
_TS = 1024


def _pool_body(x_ref, m_ref, rw1_ref, rb1_ref, rw2_ref, rb2_ref, o_ref,
               bl_ref, pooled, dsum):
    b = pl.program_id(0)
    s = pl.program_id(1)

    @pl.when(jnp.logical_and(b == 0, s == 0))
    def _():
        pooled[...] = jnp.zeros_like(pooled)
        dsum[...] = jnp.zeros_like(dsum)

    xb = x_ref[0]
    m4 = m_ref[...].astype(jnp.float32)
    row4 = jax.lax.broadcasted_iota(jnp.int32, (_B, _PS), 0)
    msel = jnp.where(row4 == b, m4, 0.0)
    lhs = jnp.concatenate(
        [msel, jnp.zeros((8 - _B, _PS), jnp.float32)], axis=0)
    pooled[...] += jax.lax.dot_general(
        lhs, xb, (((1,), (0,)), ((), ())),
        precision=jax.lax.Precision.HIGHEST,
        preferred_element_type=jnp.float32)
    msum = jnp.sum(msel)
    dsum[pl.ds(b, 1), :] += jnp.full((1, 128), msum, jnp.float32)

    @pl.when(jnp.logical_and(b == pl.num_programs(0) - 1,
                             s == pl.num_programs(1) - 1))
    def _():
        denom = jnp.maximum(dsum[:, 0:1], 1.0)
        p = pooled[...] / denom
        h = jax.lax.dot_general(
            p, rw1_ref[...], (((1,), (0,)), ((), ())),
            precision=jax.lax.Precision.HIGHEST,
            preferred_element_type=jnp.float32)
        h = jnp.maximum(h + rb1_ref[...], 0.0)
        lg = jax.lax.dot_general(
            h, rw2_ref[...], (((1,), (0,)), ((), ())),
            precision=jax.lax.Precision.HIGHEST,
            preferred_element_type=jnp.float32)
        lg = lg + rb2_ref[...]
        row = jax.lax.broadcasted_iota(jnp.int32, (8, _E), 0)
        rmax = jnp.max(lg, axis=1, keepdims=True)
        pr = jnp.exp(lg - rmax)
        probs = pr / jnp.sum(pr, axis=1, keepdims=True)
        u = jnp.sum(jnp.where(row < _B, probs, 0.0), axis=0,
                    keepdims=True) * (1.0 / _B)
        d = u - 1.0 / _E
        bl = (_BAL_W / _E) * jnp.sum(d * d)
        bl_ref[...] = jnp.full((1, 1), bl, jnp.float32)
        o_ref[...] = jnp.concatenate(
            [jnp.transpose(lg), jnp.zeros((_E, 16 - 8), jnp.float32)],
            axis=1)


_PS = 1024


def _pool_logits(hidden, mask3, rw1, rb1_2d, rw2_p, rb2_2d):
    return pl.pallas_call(
        _pool_body,
        grid=(_B, _S // _PS),
        in_specs=[
            pl.BlockSpec((1, _PS, _H), lambda b, s: (b, s, 0)),
            pl.BlockSpec((_B, _PS), lambda b, s: (0, s)),
            pl.BlockSpec((_H, _RH), lambda b, s: (0, 0)),
            pl.BlockSpec((1, _RH), lambda b, s: (0, 0)),
            pl.BlockSpec((_RH, _E), lambda b, s: (0, 0)),
            pl.BlockSpec((1, _E), lambda b, s: (0, 0)),
        ],
        out_specs=[pl.BlockSpec((_E, 16), lambda b, s: (0, 0)),
                   pl.BlockSpec((1, 1), lambda b, s: (0, 0))],
        out_shape=[jax.ShapeDtypeStruct((_E, 16), jnp.float32),
                   jax.ShapeDtypeStruct((1, 1), jnp.float32)],
        scratch_shapes=[pltpu.VMEM((8, _H), jnp.float32),
                        pltpu.VMEM((8, 128), jnp.float32)],
    )(hidden, mask3, rw1, rb1_2d, rw2_p, rb2_2d)


def _route_body(lg_hbm, rw_hbm, lg_v, rw_v):
    cid = jax.lax.axis_index("c")
    sid = jax.lax.axis_index("s")

    @pl.when(jnp.logical_and(cid == 0, sid == 0))
    def _():
        pltpu.sync_copy(lg_hbm, lg_v)
        m1 = lg_v[0]
        i1 = jnp.zeros((16,), jnp.int32)
        for e in range(1, _E):
            le = lg_v[e]
            upd = le > m1
            m1 = jnp.where(upd, le, m1)
            i1 = jnp.where(upd, e, i1)
        m2 = jnp.full((16,), _NEG, jnp.float32)
        i2 = jnp.full((16,), 16, jnp.int32)
        for e in range(_E):
            le = lg_v[e]
            upd = jnp.logical_and(i1 != e, le > m2)
            m2 = jnp.where(upd, le, m2)
            i2 = jnp.where(upd, e, i2)
        e2 = jnp.exp(m2 - m1)
        z2 = 1.0 + e2
        w1 = 1.0 / z2
        w2 = e2 / z2
        for e in range(_E):
            rw_v[e] = jnp.where(i1 == e, w1, jnp.where(i2 == e, w2, 0.0))
        pltpu.sync_copy(rw_v, rw_hbm)


def _route(lgT):
    mesh = plsc.VectorSubcoreMesh(core_axis_name="c", subcore_axis_name="s")
    run = functools.partial(
        pl.kernel,
        mesh=mesh,
        out_type=jax.ShapeDtypeStruct((_E, 16), jnp.float32),
        scratch_types=[
            pltpu.VMEM((_E, 16), jnp.float32),
            pltpu.VMEM((_E, 16), jnp.float32),
        ],
    )(_route_body)
    return run(lgT)


def _weff_body(w_ref, a_ref, b_ref, rw_ref, o_ref, rwo_ref):
    b = pl.program_id(0)
    lane_e = jax.lax.broadcasted_iota(jnp.int32, (1, _E * _R), 1) // _R
    wl = jnp.zeros((1, _E * _R), jnp.float32)
    for e in range(_E):
        wl = jnp.where(lane_e == e, rw_ref[e, b] * _SCALE, wl)
    aw = (a_ref[...] * wl).astype(jnp.bfloat16)
    bb = b_ref[...].astype(jnp.bfloat16)
    delta = jax.lax.dot_general(
        aw, bb, (((1,), (0,)), ((), ())),
        preferred_element_type=jnp.float32)
    o_ref[0] = (w_ref[...] + delta).astype(jnp.bfloat16)
    col_e = jax.lax.broadcasted_iota(jnp.int32, (1, _E), 1)
    row = jnp.zeros((1, _E), jnp.float32)
    for e in range(_E):
        row = jnp.where(col_e == e, rw_ref[e, b], row)
    rwo_ref[pl.ds(b, 1), :] = row


def _weff(w_base, a_all, b_all, rw_T):
    return pl.pallas_call(
        _weff_body,
        grid=(_B,),
        in_specs=[
            pl.BlockSpec((_H, _H), lambda b: (0, 0)),
            pl.BlockSpec((_H, _E * _R), lambda b: (0, 0)),
            pl.BlockSpec((_E * _R, _H), lambda b: (0, 0)),
            pl.BlockSpec(memory_space=pltpu.SMEM),
        ],
        out_specs=[pl.BlockSpec((1, _H, _H), lambda b: (b, 0, 0)),
                   pl.BlockSpec((_B, _E), lambda b: (0, 0))],
        out_shape=[jax.ShapeDtypeStruct((_B, _H, _H), jnp.bfloat16),
                   jax.ShapeDtypeStruct((_B, _E), jnp.float32)],
    )(w_base, a_all, b_all, rw_T)


def _moe_body(x_ref, w_ref, o_ref):
    xb = x_ref[0].astype(jnp.bfloat16)
    o_ref[0] = jax.lax.dot_general(
        xb, w_ref[0], (((1,), (0,)), ((), ())),
        preferred_element_type=jnp.float32)


def _moe_matmul(hidden, weff):
    return pl.pallas_call(
        _moe_body,
        grid=(_B, _S // _TS),
        in_specs=[
            pl.BlockSpec((1, _TS, _H), lambda b, s: (b, s, 0)),
            pl.BlockSpec((1, _H, _H), lambda b, s: (b, 0, 0)),
        ],
        out_specs=pl.BlockSpec((1, _TS, _H), lambda b, s: (b, s, 0)),
        out_shape=jax.ShapeDtypeStruct((_B, _S, _H), jnp.float32),
    )(hidden, weff)


def kernel(hidden_states, attention_mask, W_base, rW1, rb1, rW2, rb2,
           lora_A, lora_B):
    rb1_2d = rb1.reshape(1, _RH)
    rb2_2d = rb2.reshape(1, _E)

    lgT, bl11 = _pool_logits(hidden_states, attention_mask, rW1, rb1_2d,
                             rW2, rb2_2d)
    balance_loss = bl11.reshape(())

    rw_T = _route(lgT)

    a_all = jnp.transpose(lora_A, (1, 0, 2)).reshape(_H, _E * _R)
    weff, routing_weights = _weff(W_base, a_all,
                                  lora_B.reshape(_E * _R, _H), rw_T)
    out = _moe_matmul(hidden_states, weff)
    return out, routing_weights, balance_loss


# speedup vs baseline: 1.0784x; 1.0784x over previous
import functools

import jax
import jax.numpy as jnp
from jax.experimental import pallas as pl
from jax.experimental.pallas import tpu as pltpu
from jax.experimental.pallas import tpu_sc as plsc

_B, _S, _H = 4, 2048, 2048
_E, _K, _R, _RH = 8, 2, 16, 256
_SCALE = 32.0 / 16.0
_BAL_W = 0.1
_NEG = -1e30
_TS = 1024


def _pool_body(x_ref, m_ref, rw1_ref, rb1_ref, rw2_ref, rb2_ref, o_ref,
               bl_ref, pooled, dsum):
    b = pl.program_id(0)
    s = pl.program_id(1)

    @pl.when(jnp.logical_and(b == 0, s == 0))
    def _():
        pooled[...] = jnp.zeros_like(pooled)
        dsum[...] = jnp.zeros_like(dsum)

    xb = x_ref[0]
    m4 = m_ref[...].astype(jnp.float32)
    row4 = jax.lax.broadcasted_iota(jnp.int32, (_B, _PS), 0)
    msel = jnp.where(row4 == b, m4, 0.0)
    mrow = jnp.sum(msel, axis=0, keepdims=True)
    mcol = jnp.transpose(mrow)
    msum = jnp.sum(mrow)
    colsum = jnp.sum(xb * mcol, axis=0)
    pooled[pl.ds(b, 1), :] += colsum.reshape(1, _H)
    dsum[pl.ds(b, 1), :] += jnp.full((1, 128), msum, jnp.float32)

    @pl.when(jnp.logical_and(b == pl.num_programs(0) - 1,
                             s == pl.num_programs(1) - 1))
    def _():
        denom = jnp.maximum(dsum[:, 0:1], 1.0)
        p = pooled[...] / denom
        h = jax.lax.dot_general(
            p, rw1_ref[...], (((1,), (0,)), ((), ())),
            precision=jax.lax.Precision.HIGHEST,
            preferred_element_type=jnp.float32)
        h = jnp.maximum(h + rb1_ref[...], 0.0)
        lg = jax.lax.dot_general(
            h, rw2_ref[...], (((1,), (0,)), ((), ())),
            precision=jax.lax.Precision.HIGHEST,
            preferred_element_type=jnp.float32)
        lg = lg + rb2_ref[...]
        row = jax.lax.broadcasted_iota(jnp.int32, (8, _E), 0)
        rmax = jnp.max(lg, axis=1, keepdims=True)
        pr = jnp.exp(lg - rmax)
        probs = pr / jnp.sum(pr, axis=1, keepdims=True)
        u = jnp.sum(jnp.where(row < _B, probs, 0.0), axis=0,
                    keepdims=True) * (1.0 / _B)
        d = u - 1.0 / _E
        bl = (_BAL_W / _E) * jnp.sum(d * d)
        bl_ref[...] = jnp.full((1, 1), bl, jnp.float32)
        o_ref[...] = jnp.concatenate(
            [jnp.transpose(lg), jnp.zeros((_E, 16 - 8), jnp.float32)],
            axis=1)


_PS = 1024


def _pool_logits(hidden, mask3, rw1, rb1_2d, rw2_p, rb2_2d):
    return pl.pallas_call(
        _pool_body,
        grid=(_B, _S // _PS),
        in_specs=[
            pl.BlockSpec((1, _PS, _H), lambda b, s: (b, s, 0)),
            pl.BlockSpec((_B, _PS), lambda b, s: (0, s)),
            pl.BlockSpec((_H, _RH), lambda b, s: (0, 0)),
            pl.BlockSpec((1, _RH), lambda b, s: (0, 0)),
            pl.BlockSpec((_RH, _E), lambda b, s: (0, 0)),
            pl.BlockSpec((1, _E), lambda b, s: (0, 0)),
        ],
        out_specs=[pl.BlockSpec((_E, 16), lambda b, s: (0, 0)),
                   pl.BlockSpec((1, 1), lambda b, s: (0, 0))],
        out_shape=[jax.ShapeDtypeStruct((_E, 16), jnp.float32),
                   jax.ShapeDtypeStruct((1, 1), jnp.float32)],
        scratch_shapes=[pltpu.VMEM((8, _H), jnp.float32),
                        pltpu.VMEM((8, 128), jnp.float32)],
    )(hidden, mask3, rw1, rb1_2d, rw2_p, rb2_2d)


def _route_body(lg_hbm, rw_hbm, lg_v, rw_v):
    cid = jax.lax.axis_index("c")
    sid = jax.lax.axis_index("s")

    @pl.when(jnp.logical_and(cid == 0, sid == 0))
    def _():
        pltpu.sync_copy(lg_hbm, lg_v)
        m1 = lg_v[0]
        i1 = jnp.zeros((16,), jnp.int32)
        for e in range(1, _E):
            le = lg_v[e]
            upd = le > m1
            m1 = jnp.where(upd, le, m1)
            i1 = jnp.where(upd, e, i1)
        m2 = jnp.full((16,), _NEG, jnp.float32)
        i2 = jnp.full((16,), 16, jnp.int32)
        for e in range(_E):
            le = lg_v[e]
            upd = jnp.logical_and(i1 != e, le > m2)
            m2 = jnp.where(upd, le, m2)
            i2 = jnp.where(upd, e, i2)
        e2 = jnp.exp(m2 - m1)
        z2 = 1.0 + e2
        w1 = 1.0 / z2
        w2 = e2 / z2
        for e in range(_E):
            rw_v[e] = jnp.where(i1 == e, w1, jnp.where(i2 == e, w2, 0.0))
        pltpu.sync_copy(rw_v, rw_hbm)


def _route(lgT):
    mesh = plsc.VectorSubcoreMesh(core_axis_name="c", subcore_axis_name="s")
    run = functools.partial(
        pl.kernel,
        mesh=mesh,
        out_type=jax.ShapeDtypeStruct((_E, 16), jnp.float32),
        scratch_types=[
            pltpu.VMEM((_E, 16), jnp.float32),
            pltpu.VMEM((_E, 16), jnp.float32),
        ],
    )(_route_body)
    return run(lgT)


def _weff_body(w_ref, a_ref, b_ref, rw_ref, o_ref, rwo_ref):
    b = pl.program_id(0)
    lane_e = jax.lax.broadcasted_iota(jnp.int32, (1, _E * _R), 1) // _R
    wl = jnp.zeros((1, _E * _R), jnp.float32)
    for e in range(_E):
        wl = jnp.where(lane_e == e, rw_ref[e, b] * _SCALE, wl)
    aw = (a_ref[...] * wl).astype(jnp.bfloat16)
    bb = b_ref[...].astype(jnp.bfloat16)
    delta = jax.lax.dot_general(
        aw, bb, (((1,), (0,)), ((), ())),
        preferred_element_type=jnp.float32)
    o_ref[0] = (w_ref[...] + delta).astype(jnp.bfloat16)
    col_e = jax.lax.broadcasted_iota(jnp.int32, (1, _E), 1)
    row = jnp.zeros((1, _E), jnp.float32)
    for e in range(_E):
        row = jnp.where(col_e == e, rw_ref[e, b], row)
    rwo_ref[pl.ds(b, 1), :] = row


def _weff(w_base, a_all, b_all, rw_T):
    return pl.pallas_call(
        _weff_body,
        grid=(_B,),
        in_specs=[
            pl.BlockSpec((_H, _H), lambda b: (0, 0)),
            pl.BlockSpec((_H, _E * _R), lambda b: (0, 0)),
            pl.BlockSpec((_E * _R, _H), lambda b: (0, 0)),
            pl.BlockSpec(memory_space=pltpu.SMEM),
        ],
        out_specs=[pl.BlockSpec((1, _H, _H), lambda b: (b, 0, 0)),
                   pl.BlockSpec((_B, _E), lambda b: (0, 0))],
        out_shape=[jax.ShapeDtypeStruct((_B, _H, _H), jnp.bfloat16),
                   jax.ShapeDtypeStruct((_B, _E), jnp.float32)],
    )(w_base, a_all, b_all, rw_T)


def _moe_body(x_ref, w_ref, o_ref):
    xb = x_ref[0].astype(jnp.bfloat16)
    o_ref[0] = jax.lax.dot_general(
        xb, w_ref[0], (((1,), (0,)), ((), ())),
        preferred_element_type=jnp.float32)


def _moe_matmul(hidden, weff):
    return pl.pallas_call(
        _moe_body,
        grid=(_B, _S // _TS),
        in_specs=[
            pl.BlockSpec((1, _TS, _H), lambda b, s: (b, s, 0)),
            pl.BlockSpec((1, _H, _H), lambda b, s: (b, 0, 0)),
        ],
        out_specs=pl.BlockSpec((1, _TS, _H), lambda b, s: (b, s, 0)),
        out_shape=jax.ShapeDtypeStruct((_B, _S, _H), jnp.float32),
    )(hidden, weff)


def kernel(hidden_states, attention_mask, W_base, rW1, rb1, rW2, rb2,
           lora_A, lora_B):
    rb1_2d = rb1.reshape(1, _RH)
    rb2_2d = rb2.reshape(1, _E)

    lgT, bl11 = _pool_logits(hidden_states, attention_mask, rW1, rb1_2d,
                             rW2, rb2_2d)
    balance_loss = bl11.reshape(())

    rw_T = _route(lgT)

    a_all = jnp.transpose(lora_A, (1, 0, 2)).reshape(_H, _E * _R)
    weff, routing_weights = _weff(W_base, a_all,
                                  lora_B.reshape(_E * _R, _H), rw_T)
    out = _moe_matmul(hidden_states, weff)
    return out, routing_weights, balance_loss
